# baseline (speedup 1.0000x reference)
import functools

import jax
import jax.numpy as jnp
from jax import lax
from jax.experimental import pallas as pl
from jax.experimental.pallas import tpu as pltpu
from jax.experimental.pallas import tpu_sc as plsc

B = 16384
L = 200
V = 17
D = 16
LP = 256
NC, NS = 2, 16
NW = NC * NS
BSC = 8192
BTC = B - BSC
TCBLK = 1024
ROWS = BSC // NW
CROWS = 128
NCHUNK = ROWS // CROWS
NSLICE = 13

_mesh = plsc.VectorSubcoreMesh(core_axis_name="c", subcore_axis_name="s",
                               num_cores=NC, num_subcores=NS)


@functools.partial(
    pl.kernel,
    out_type=jax.ShapeDtypeStruct((NW, V * LP), jnp.float32),
    mesh=_mesh,
    compiler_params=pltpu.CompilerParams(needs_layout_passes=False),
    scratch_types=[
        pltpu.VMEM((CROWS, L), jnp.int32),
        pltpu.VMEM((CROWS, L), jnp.int32),
        pltpu.VMEM((V * LP,), jnp.float32),
        pltpu.SemaphoreType.DMA,
        pltpu.SemaphoreType.DMA,
    ],
)
def _sc_hist(x_hbm, out_hbm, xb0, xb1, cnt, sem0, sem1):
    wid = lax.axis_index("s") * NC + lax.axis_index("c")
    bufs = (xb0, xb1)
    sems = (sem0, sem1)

    @plsc.parallel_loop(0, V * LP // 16, unroll=4)
    def _(j):
        cnt[pl.ds(j * 16, 16)] = jnp.zeros((16,), jnp.float32)

    row0 = wid * ROWS

    def start(k):
        return pltpu.async_copy(
            x_hbm.at[pl.ds(row0 + k * CROWS, CROWS)], bufs[k % 2], sems[k % 2])

    ones = jnp.ones((16,), jnp.float32)
    iota = lax.iota(jnp.int32, 16)
    tail_mask = iota >= 8
    lvecs = [iota + (c * 16 if c < NSLICE - 1 else L - 16)
             for c in range(NSLICE)]

    descs = [start(0), start(1)] + [None] * (NCHUNK - 2)

    for k in range(NCHUNK):
        descs[k].wait()
        buf = bufs[k % 2]

        @plsc.parallel_loop(0, CROWS, unroll=4)
        def _(r):
            for c in range(NSLICE):
                off = c * 16 if c < NSLICE - 1 else L - 16
                v = buf[r, pl.ds(off, 16)]
                idx = lax.shift_left(v, 8) | lvecs[c]
                if c < NSLICE - 1:
                    plsc.addupdate_scatter(cnt, [idx], ones)
                else:
                    plsc.addupdate_scatter(cnt, [idx], ones, mask=tail_mask)

        if k + 2 < NCHUNK:
            descs[k + 2] = start(k + 2)

    pltpu.sync_copy(cnt, out_hbm.at[wid])


def _tc_hist_body(x_ref, out_ref):
    @pl.when(pl.program_id(0) == 0)
    def _():
        out_ref[...] = jnp.zeros((2 * D, LP), jnp.float32)

    xb = x_ref[...]
    for v in range(V):
        row = jnp.sum(jnp.where(xb == v, 1.0, 0.0), axis=0)
        out_ref[v, :L] += row


def _tc_body(cnt_ref, tchist_ref, embed_ref, fcw_ref, bias_ref, out_ref):
    ct = jnp.sum(cnt_ref[...], axis=0).reshape(V, LP)
    ct = ct + tchist_ref[:V]
    m = lax.dot_general(ct, embed_ref[...],
                        (((0,), (0,)), ((), ())),
                        preferred_element_type=jnp.float32)
    out = lax.dot_general(m * (1.0 / B), fcw_ref[...],
                          (((1,), (1,)), ((), ())),
                          preferred_element_type=jnp.float32)
    out_ref[...] = out[:L] + bias_ref[...]


def kernel(x, embed_weight, fc_weight, fc_bias):
    xi = x.astype(jnp.int32)
    tchist = pl.pallas_call(
        _tc_hist_body,
        grid=(BTC // TCBLK,),
        in_specs=[pl.BlockSpec((TCBLK, L), lambda i: (i, 0))],
        out_specs=pl.BlockSpec((2 * D, LP), lambda i: (0, 0)),
        out_shape=jax.ShapeDtypeStruct((2 * D, LP), jnp.float32),
    )(xi)
    xsc = lax.slice(xi, (BTC, 0), (B, L))
    counts = _sc_hist(xsc)
    out = pl.pallas_call(
        _tc_body,
        out_shape=jax.ShapeDtypeStruct((L, V), jnp.float32),
    )(counts, tchist, embed_weight, fc_weight, fc_bias.reshape(1, V))
    return out


# speedup vs baseline: 1.0036x; 1.0036x over previous
import functools

import jax
import jax.numpy as jnp
from jax import lax
from jax.experimental import pallas as pl
from jax.experimental.pallas import tpu as pltpu
from jax.experimental.pallas import tpu_sc as plsc

B = 16384
L = 200
V = 17
D = 16
LP = 256
NC, NS = 2, 16
NW = NC * NS
PR = B // 4
WPR = PR // NW
HPR = WPR // 2
TCBLK = 1024
HI_W = L - 128

_mesh = plsc.VectorSubcoreMesh(core_axis_name="c", subcore_axis_name="s",
                               num_cores=NC, num_subcores=NS)


def _tc_pack_body(x_ref, lo_ref, hi_ref):
    xb = x_ref[...].reshape(TCBLK // 4, 4, L)
    r0, r1, r2, r3 = xb[:, 0], xb[:, 1], xb[:, 2], xb[:, 3]
    w = (r0 | lax.shift_left(r1, 8) | lax.shift_left(r2, 16)
         | lax.shift_left(r3, 24))
    lo_ref[...] = w[:, :128]
    hi_ref[...] = jnp.concatenate(
        [w[:, 128:], jnp.zeros((TCBLK // 4, 128 - HI_W), jnp.int32)], axis=1)


@functools.partial(
    pl.kernel,
    out_type=jax.ShapeDtypeStruct((NW, V * LP), jnp.float32),
    mesh=_mesh,
    compiler_params=pltpu.CompilerParams(needs_layout_passes=False),
    scratch_types=[
        pltpu.VMEM((HPR, 128), jnp.int32),
        pltpu.VMEM((HPR, 128), jnp.int32),
        pltpu.VMEM((HPR, 128), jnp.int32),
        pltpu.VMEM((HPR, 128), jnp.int32),
        pltpu.VMEM((V * LP,), jnp.float32),
        pltpu.SemaphoreType.DMA,
        pltpu.SemaphoreType.DMA,
        pltpu.SemaphoreType.DMA,
        pltpu.SemaphoreType.DMA,
    ],
)
def _sc_hist(lo_hbm, hi_hbm, out_hbm, lo0, lo1, hi0, hi1, cnt,
             sl0, sl1, sh0, sh1):
    wid = lax.axis_index("s") * NC + lax.axis_index("c")

    @plsc.parallel_loop(0, V * LP // 16, unroll=4)
    def _(j):
        cnt[pl.ds(j * 16, 16)] = jnp.zeros((16,), jnp.float32)

    row0 = wid * WPR
    descs = [
        pltpu.async_copy(lo_hbm.at[pl.ds(row0, HPR)], lo0, sl0),
        pltpu.async_copy(hi_hbm.at[pl.ds(row0, HPR)], hi0, sh0),
        pltpu.async_copy(lo_hbm.at[pl.ds(row0 + HPR, HPR)], lo1, sl1),
        pltpu.async_copy(hi_hbm.at[pl.ds(row0 + HPR, HPR)], hi1, sh1),
    ]

    ones = jnp.ones((16,), jnp.float32)
    iota = lax.iota(jnp.int32, 16)
    himask = iota < 8
    lolv = [iota + s * 16 for s in range(8)]
    hilv = [iota + 128 + s * 16 for s in range(5)]

    def scat4(w, lv, mask=None):
        for kk in range(4):
            if kk == 0:
                v = w & 0xFF
            elif kk < 3:
                v = lax.shift_right_logical(w, 8 * kk) & 0xFF
            else:
                v = lax.shift_right_logical(w, 24)
            idx = lax.shift_left(v, 8) | lv
            if mask is None:
                plsc.addupdate_scatter(cnt, [idx], ones)
            else:
                plsc.addupdate_scatter(cnt, [idx], ones, mask=mask)

    for half, (lob, hib) in enumerate(((lo0, hi0), (lo1, hi1))):
        descs[2 * half].wait()
        descs[2 * half + 1].wait()

        @plsc.parallel_loop(0, HPR, unroll=2)
        def _(r):
            for s in range(8):
                scat4(lob[r, pl.ds(s * 16, 16)], lolv[s])
            for s in range(4):
                scat4(hib[r, pl.ds(s * 16, 16)], hilv[s])
            scat4(hib[r, pl.ds(64, 16)], hilv[4], mask=himask)

    pltpu.sync_copy(cnt, out_hbm.at[wid])


def _tc_body(cnt_ref, embed_ref, fcw_ref, bias_ref, out_ref):
    ct = jnp.sum(cnt_ref[...], axis=0).reshape(V, LP)
    m = lax.dot_general(ct, embed_ref[...],
                        (((0,), (0,)), ((), ())),
                        preferred_element_type=jnp.float32)
    out = lax.dot_general(m * (1.0 / B), fcw_ref[...],
                          (((1,), (1,)), ((), ())),
                          preferred_element_type=jnp.float32)
    out_ref[...] = out[:L] + bias_ref[...]


def kernel(x, embed_weight, fc_weight, fc_bias):
    xi = x.astype(jnp.int32)
    lo, hi = pl.pallas_call(
        _tc_pack_body,
        grid=(B // TCBLK,),
        in_specs=[pl.BlockSpec((TCBLK, L), lambda i: (i, 0))],
        out_specs=[pl.BlockSpec((TCBLK // 4, 128), lambda i: (i, 0)),
                   pl.BlockSpec((TCBLK // 4, 128), lambda i: (i, 0))],
        out_shape=[jax.ShapeDtypeStruct((PR, 128), jnp.int32),
                   jax.ShapeDtypeStruct((PR, 128), jnp.int32)],
    )(xi)
    counts = _sc_hist(lo, hi)
    out = pl.pallas_call(
        _tc_body,
        out_shape=jax.ShapeDtypeStruct((L, V), jnp.float32),
    )(counts, embed_weight, fc_weight, fc_bias.reshape(1, V))
    return out


# speedup vs baseline: 1.2574x; 1.2529x over previous
import functools

import jax
import jax.numpy as jnp
from jax import lax
from jax.experimental import pallas as pl
from jax.experimental.pallas import tpu as pltpu
from jax.experimental.pallas import tpu_sc as plsc

B = 16384
L = 200
V = 17
D = 16
LP = 256
NC, NS = 2, 16
NW = NC * NS
ROWS = B // NW
CROWS = 128
NCHUNK = ROWS // CROWS
NSLICE = 13

_mesh = plsc.VectorSubcoreMesh(core_axis_name="c", subcore_axis_name="s",
                               num_cores=NC, num_subcores=NS)


@functools.partial(
    pl.kernel,
    out_type=jax.ShapeDtypeStruct((NW, V * LP), jnp.float32),
    mesh=_mesh,
    compiler_params=pltpu.CompilerParams(needs_layout_passes=False),
    scratch_types=[
        pltpu.VMEM((CROWS, L), jnp.int32),
        pltpu.VMEM((CROWS, L), jnp.int32),
        pltpu.VMEM((V * LP,), jnp.float32),
        pltpu.SemaphoreType.DMA,
        pltpu.SemaphoreType.DMA,
    ],
)
def _sc_hist(x_hbm, out_hbm, xb0, xb1, cnt, sem0, sem1):
    wid = lax.axis_index("s") * NC + lax.axis_index("c")
    bufs = (xb0, xb1)
    sems = (sem0, sem1)

    @plsc.parallel_loop(0, V * LP // 16, unroll=4)
    def _(j):
        cnt[pl.ds(j * 16, 16)] = jnp.zeros((16,), jnp.float32)

    row0 = wid * ROWS

    def start(k):
        return pltpu.async_copy(
            x_hbm.at[pl.ds(row0 + k * CROWS, CROWS)], bufs[k % 2], sems[k % 2])

    ones = jnp.ones((16,), jnp.float32)
    iota = lax.iota(jnp.int32, 16)
    tail_mask = iota >= 8
    lvecs = [iota + (c * 16 if c < NSLICE - 1 else L - 16)
             for c in range(NSLICE)]

    descs = [start(0), start(1), None, None]

    for k in range(NCHUNK):
        descs[k].wait()
        buf = bufs[k % 2]

        @plsc.parallel_loop(0, CROWS, unroll=4)
        def _(r):
            for c in range(NSLICE):
                off = c * 16 if c < NSLICE - 1 else L - 16
                v = buf[r, pl.ds(off, 16)]
                idx = lax.shift_left(v, 8) | lvecs[c]
                if c < NSLICE - 1:
                    plsc.addupdate_scatter(cnt, [idx], ones)
                else:
                    plsc.addupdate_scatter(cnt, [idx], ones, mask=tail_mask)

        if k + 2 < NCHUNK:
            descs[k + 2] = start(k + 2)

    pltpu.sync_copy(cnt, out_hbm.at[wid])


def _tc_body(cnt_ref, embed_ref, fcw_ref, bias_ref, out_ref):
    ct = jnp.sum(cnt_ref[...], axis=0).reshape(V, LP)
    m = lax.dot_general(ct, embed_ref[...],
                        (((0,), (0,)), ((), ())),
                        preferred_element_type=jnp.float32)
    out = lax.dot_general(m * (1.0 / B), fcw_ref[...],
                          (((1,), (1,)), ((), ())),
                          preferred_element_type=jnp.float32)
    out_ref[...] = out[:L] + bias_ref[...]


def kernel(x, embed_weight, fc_weight, fc_bias):
    counts = _sc_hist(x.astype(jnp.int32))
    out = pl.pallas_call(
        _tc_body,
        out_shape=jax.ShapeDtypeStruct((L, V), jnp.float32),
    )(counts, embed_weight, fc_weight, fc_bias.reshape(1, V))
    return out
